# Initial kernel scaffold; baseline (speedup 1.0000x reference)
import jax
import jax.numpy as jnp
from jax import lax
from jax.experimental import pallas as pl
from jax.experimental.pallas import tpu as pltpu
from jax.experimental.pallas import tpu_sc as plsc

NUM_CORES = 2
NUM_SUBCORES = 16
LANES = 16
NUM_WORKERS = NUM_CORES * NUM_SUBCORES

BATCH = 16384
HIST = 50
EMBED_DIM = 64
DREGS = EMBED_DIM // LANES

BAGS_PER_WORKER = BATCH // NUM_WORKERS
CHUNK = 16
NUM_CHUNKS = BAGS_PER_WORKER // CHUNK
NBUF = 2


def _body(bi_hbm, table_hbm, out_hbm, idx_v, rows_v, out_v, gsems):
    wid = lax.axis_index("s") * NUM_CORES + lax.axis_index("c")
    bag0 = wid * BAGS_PER_WORKER

    def start_chunk(g, slot):
        base = bag0 + g * CHUNK
        pltpu.sync_copy(bi_hbm.at[pl.ds(base, CHUNK)], idx_v.at[slot])
        pltpu.async_copy(table_hbm.at[idx_v.at[slot]], rows_v.at[slot],
                         gsems.at[slot])

    def reduce_chunk(g, slot):
        scale = jnp.full((LANES,), 1.0 / HIST, dtype=jnp.float32)
        for c in range(CHUNK):
            accs = [rows_v[slot, c, 0, pl.ds(r * LANES, LANES)]
                    for r in range(DREGS)]
            for j in range(1, HIST):
                for r in range(DREGS):
                    accs[r] = accs[r] + rows_v[slot, c, j,
                                               pl.ds(r * LANES, LANES)]
            for r in range(DREGS):
                out_v[c, pl.ds(r * LANES, LANES)] = accs[r] * scale
        pltpu.sync_copy(out_v, out_hbm.at[pl.ds(bag0 + g * CHUNK, CHUNK)])

    for b in range(NBUF):
        start_chunk(b, b)

    for g in range(NUM_CHUNKS):
        slot = g % NBUF
        pltpu.make_async_copy(table_hbm.at[idx_v.at[slot]], rows_v.at[slot],
                              gsems.at[slot]).wait()
        reduce_chunk(g, slot)
        if g + NBUF < NUM_CHUNKS:
            start_chunk(g + NBUF, slot)


@jax.jit
def _embedding_bag_mean(batch_input, table):
    mesh = plsc.VectorSubcoreMesh(core_axis_name="c", subcore_axis_name="s")
    kern = pl.kernel(
        _body,
        out_type=jax.ShapeDtypeStruct((BATCH, EMBED_DIM), jnp.float32),
        mesh=mesh,
        scratch_types=[
            pltpu.VMEM((NBUF, CHUNK, HIST), jnp.int32),
            pltpu.VMEM((NBUF, CHUNK, HIST, EMBED_DIM), jnp.float32),
            pltpu.VMEM((CHUNK, EMBED_DIM), jnp.float32),
            pltpu.SemaphoreType.DMA((NBUF,)),
        ],
    )
    return kern(batch_input, table)


def kernel(batch_input, table):
    return _embedding_bag_mean(batch_input, table)


# speedup vs baseline: 2.7082x; 2.7082x over previous
import jax
import jax.numpy as jnp
from jax import lax
from jax.experimental import pallas as pl
from jax.experimental.pallas import tpu as pltpu
from jax.experimental.pallas import tpu_sc as plsc

NUM_CORES = 2
NUM_SUBCORES = 16
LANES = 16
NUM_WORKERS = NUM_CORES * NUM_SUBCORES

BATCH = 16384
HIST = 50
EMBED_DIM = 64
DREGS = EMBED_DIM // LANES

PAIR = 2
IDXROW = PAIR * HIST
BAGS_PER_WORKER = BATCH // NUM_WORKERS
CHUNK = 16
K = CHUNK // PAIR
NUM_CHUNKS = BAGS_PER_WORKER // CHUNK
NBUF = 2


def _body(bi_hbm, table_hbm, out_hbm, idx_v, rows_v, out_v, gsems):
    wid = lax.axis_index("s") * NUM_CORES + lax.axis_index("c")
    bag0 = wid * BAGS_PER_WORKER
    row0 = bag0 // PAIR
    scale = jnp.full((LANES,), 1.0 / HIST, dtype=jnp.float32)

    def start_chunk(g, slot):
        base = pl.multiple_of(row0 + g * K, 8)
        pltpu.sync_copy(bi_hbm.at[pl.ds(base, K)], idx_v.at[slot])
        for j in range(K):
            pltpu.async_copy(table_hbm.at[idx_v.at[slot, j]],
                             rows_v.at[slot, j], gsems.at[slot])

    def process_chunk(g, slot):
        for j in range(K):
            pltpu.make_async_copy(table_hbm.at[idx_v.at[slot, j]],
                                  rows_v.at[slot, j], gsems.at[slot]).wait()

        def pair_body(jr, carry):
            for half in range(PAIR):
                accs = [rows_v[slot, jr, half * HIST, pl.ds(r * LANES, LANES)]
                        for r in range(DREGS)]
                for j in range(1, HIST):
                    for r in range(DREGS):
                        accs[r] = accs[r] + rows_v[slot, jr, half * HIST + j,
                                                   pl.ds(r * LANES, LANES)]
                for r in range(DREGS):
                    out_v[jr * PAIR + half,
                          pl.ds(r * LANES, LANES)] = accs[r] * scale
            return carry

        lax.fori_loop(0, K, pair_body, 0, unroll=False)
        obase = pl.multiple_of(bag0 + g * CHUNK, 8)
        pltpu.sync_copy(out_v, out_hbm.at[pl.ds(obase, CHUNK)])

        @pl.when(g + NBUF < NUM_CHUNKS)
        def _():
            start_chunk(g + NBUF, slot)

    for b in range(NBUF):
        start_chunk(b, b)

    def outer(gg, carry):
        for b in range(NBUF):
            process_chunk(gg * NBUF + b, b)
        return carry

    lax.fori_loop(0, NUM_CHUNKS // NBUF, outer, 0, unroll=False)


@jax.jit
def _embedding_bag_mean(batch_input, table):
    mesh = plsc.VectorSubcoreMesh(core_axis_name="c", subcore_axis_name="s")
    kern = pl.kernel(
        _body,
        out_type=jax.ShapeDtypeStruct((BATCH, EMBED_DIM), jnp.float32),
        mesh=mesh,
        compiler_params=pltpu.CompilerParams(use_tc_tiling_on_sc=False),
        scratch_types=[
            pltpu.VMEM((NBUF, K, IDXROW), jnp.int32),
            pltpu.VMEM((NBUF, K, IDXROW, EMBED_DIM), jnp.float32),
            pltpu.VMEM((CHUNK, EMBED_DIM), jnp.float32),
            pltpu.SemaphoreType.DMA((NBUF,)),
        ],
    )
    return kern(batch_input.reshape(BATCH // PAIR, PAIR * HIST), table)


def kernel(batch_input, table):
    return _embedding_bag_mean(batch_input, table)
